# baseline (speedup 1.0000x reference)
import functools

import jax
import jax.numpy as jnp
from jax import lax
from jax.experimental import pallas as pl
from jax.experimental.pallas import tpu as pltpu
from jax.experimental.pallas import tpu_sc as plsc

N = 10000
E = 320000
D = 128
NEG_SLOPE = 0.2

NC = 2
NS = 16
NW = NC * NS
EPW = E // NW
CHUNK = 40
NCHUNKS = EPW // CHUNK
ROWS_PER_TILE = 624
TAIL_ROW0 = NS * ROWS_PER_TILE
TAIL_ROWS = N - TAIL_ROW0


def _mm_body(h_ref, wl_ref, wr_ref, xl_ref, xr_ref):
    h = h_ref[...]
    xl_ref[...] = jnp.dot(h, wl_ref[...], preferred_element_type=jnp.float32)
    xr_ref[...] = jnp.dot(h, wr_ref[...], preferred_element_type=jnp.float32)


_mm_call = pl.pallas_call(
    _mm_body,
    out_shape=[jax.ShapeDtypeStruct((N, D), jnp.float32)] * 2,
)


def _edge_body(xl_hbm, xr_hbm, src_hbm, dst_hbm, att_hbm,
               out_hbm, den_hbm,
               xlrows, xrrows, srcv, dstv, denv, attv, exb,
               acc_sh, sem1, sem2, sem3, sem4):
    cid = lax.axis_index("c")
    sid = lax.axis_index("s")
    wid = cid * NS + sid

    pltpu.sync_copy(att_hbm, attv)
    att_vs = [attv[pl.ds(16 * k, 16)] for k in range(8)]

    zero16 = jnp.zeros((16,), jnp.float32)
    lane = lax.iota(jnp.int32, 16)
    lane0 = lane == 0

    def zden_body(j, _):
        denv[pl.ds(16 * j, 16)] = zero16
        return 0

    lax.fori_loop(0, N // 16, zden_body, 0)

    def zrow_body(r, _):
        for k in range(8):
            xlrows[0, r, pl.ds(16 * k, 16)] = zero16
        return 0

    lax.fori_loop(0, CHUNK, zrow_body, 0)
    r0 = sid * ROWS_PER_TILE

    def zcopy_body(j, _):
        pltpu.sync_copy(xlrows.at[0],
                        acc_sh.at[pl.ds(r0 + j * CHUNK, CHUNK)])
        return 0

    lax.fori_loop(0, 15, zcopy_body, 0)
    pltpu.sync_copy(xlrows.at[0, pl.ds(0, 24)],
                    acc_sh.at[pl.ds(r0 + 600, 24)])

    @pl.when(sid == NS - 1)
    def _zero_tail():
        pltpu.sync_copy(xlrows.at[0, pl.ds(0, TAIL_ROWS)],
                        acc_sh.at[pl.ds(TAIL_ROW0, TAIL_ROWS)])

    plsc.subcore_barrier()

    pltpu.sync_copy(src_hbm.at[wid, 0], srcv.at[0])
    pltpu.sync_copy(dst_hbm.at[wid, 0], dstv.at[0])
    pltpu.sync_copy(src_hbm.at[wid, 1], srcv.at[1])
    pltpu.sync_copy(dst_hbm.at[wid, 1], dstv.at[1])
    pltpu.async_copy(xl_hbm.at[srcv.at[0]], xlrows.at[0], sem1)
    pltpu.async_copy(xr_hbm.at[dstv.at[0]], xrrows.at[0], sem2)

    def chunk_body(c, _):
        p = lax.rem(c, 2)
        slot = lax.rem(c, 3)
        slot1 = lax.rem(c + 1, 3)
        slot2 = lax.rem(c + 2, 3)

        @pl.when(c + 2 < NCHUNKS)
        def _idx_prefetch():
            pltpu.async_copy(src_hbm.at[wid, c + 2], srcv.at[slot2], sem4)
            pltpu.async_copy(dst_hbm.at[wid, c + 2], dstv.at[slot2], sem4)

        pltpu.make_async_copy(
            xl_hbm.at[srcv.at[slot]], xlrows.at[p], sem1).wait()
        pltpu.make_async_copy(
            xr_hbm.at[dstv.at[slot]], xrrows.at[p], sem2).wait()

        @pl.when(c > 0)
        def _scatter_wait():
            pltpu.make_async_copy(
                xlrows.at[1 - p], acc_sh.at[dstv.at[lax.rem(c + 2, 3)]],
                sem3).wait()

        @pl.when(c + 1 < NCHUNKS)
        def _row_prefetch():
            @pl.when(c > 0)
            def _idx_wait():
                pltpu.make_async_copy(
                    src_hbm.at[wid, c + 1], srcv.at[slot1], sem4).wait()
                pltpu.make_async_copy(
                    dst_hbm.at[wid, c + 1], dstv.at[slot1], sem4).wait()

            pltpu.async_copy(xl_hbm.at[srcv.at[slot1]], xlrows.at[1 - p], sem1)
            pltpu.async_copy(xr_hbm.at[dstv.at[slot1]], xrrows.at[1 - p], sem2)

        @plsc.parallel_loop(0, CHUNK, unroll=4)
        def edge_body(e):
            acc = jnp.zeros((16,), jnp.float32)
            xls = []
            for k in range(8):
                xlk = xlrows[p, e, pl.ds(16 * k, 16)]
                xls.append(xlk)
                m = xlk + xrrows[p, e, pl.ds(16 * k, 16)]
                m = jnp.maximum(m, NEG_SLOPE * m)
                acc = acc + m * att_vs[k]
            for s in (1, 2, 4, 8):
                acc = acc + lax.gather(
                    acc, (lane ^ s)[:, None],
                    lax.GatherDimensionNumbers(
                        offset_dims=(), collapsed_slice_dims=(0,),
                        start_index_map=(0,)),
                    slice_sizes=(1,),
                    mode=lax.GatherScatterMode.PROMISE_IN_BOUNDS)
            ex16 = jnp.exp(acc)
            plsc.store_scatter(
                exb, [jnp.full((16,), e, jnp.int32)], ex16, mask=lane0)
            for k in range(8):
                xlrows[p, e, pl.ds(16 * k, 16)] = xls[k] * ex16

        for off, msk in ((0, None), (16, None), (24, lane >= 8)):
            ex16 = exb[pl.ds(off, 16)]
            dst16 = dstv[slot, pl.ds(off, 16)]
            plsc.addupdate_scatter(denv, [dst16], ex16, mask=msk)

        pltpu.async_copy(xlrows.at[p], acc_sh.at[dstv.at[slot]], sem3,
                         add=True)
        return 0

    lax.fori_loop(0, NCHUNKS, chunk_body, 0)

    pltpu.make_async_copy(
        xlrows.at[(NCHUNKS - 1) % 2],
        acc_sh.at[dstv.at[(NCHUNKS - 1) % 3]], sem3).wait()

    pltpu.sync_copy(denv, den_hbm.at[wid])

    plsc.subcore_barrier()

    def fcopy_body(j, _):
        rr = r0 + j * CHUNK
        pltpu.sync_copy(acc_sh.at[pl.ds(rr, CHUNK)], xlrows.at[0])
        pltpu.sync_copy(xlrows.at[0], out_hbm.at[cid, pl.ds(rr, CHUNK)])
        return 0

    lax.fori_loop(0, 15, fcopy_body, 0)
    pltpu.sync_copy(acc_sh.at[pl.ds(r0 + 600, 24)],
                    xlrows.at[0, pl.ds(0, 24)])
    pltpu.sync_copy(xlrows.at[0, pl.ds(0, 24)],
                    out_hbm.at[cid, pl.ds(r0 + 600, 24)])

    @pl.when(sid == NS - 1)
    def _flush_tail():
        pltpu.sync_copy(acc_sh.at[pl.ds(TAIL_ROW0, TAIL_ROWS)],
                        xlrows.at[0, pl.ds(0, TAIL_ROWS)])
        pltpu.sync_copy(xlrows.at[0, pl.ds(0, TAIL_ROWS)],
                        out_hbm.at[cid, pl.ds(TAIL_ROW0, TAIL_ROWS)])


@functools.cache
def _make_edge_call():
  return pl.kernel(
    _edge_body,
    out_type=[
        jax.ShapeDtypeStruct((NC, N, D), jnp.float32),
        jax.ShapeDtypeStruct((NW, N), jnp.float32),
    ],
    mesh=plsc.VectorSubcoreMesh(
        core_axis_name="c", subcore_axis_name="s", num_cores=NC,
        num_subcores=NS),
    compiler_params=pltpu.CompilerParams(needs_layout_passes=False),
    scratch_types=[
        pltpu.VMEM((2, CHUNK, D), jnp.float32),
        pltpu.VMEM((2, CHUNK, D), jnp.float32),
        pltpu.VMEM((3, CHUNK), jnp.int32),
        pltpu.VMEM((3, CHUNK), jnp.int32),
        pltpu.VMEM((N,), jnp.float32),
        pltpu.VMEM((D,), jnp.float32),
        pltpu.VMEM((CHUNK,), jnp.float32),
        pltpu.VMEM_SHARED((N, D), jnp.float32),
        pltpu.SemaphoreType.DMA,
        pltpu.SemaphoreType.DMA,
        pltpu.SemaphoreType.DMA,
        pltpu.SemaphoreType.DMA,
    ],
  )


def _fin_body(op_ref, dp_ref, b_ref, o_ref):
    num = op_ref[0] + op_ref[1]
    den = jnp.sum(dp_ref[...], axis=0)[:, None]
    h0 = num / (den + 1e-16) + b_ref[...][None, :]
    mu = jnp.mean(h0, axis=0, keepdims=True)
    hc = h0 - mu
    s = jnp.mean(jnp.sum(hc * hc, axis=-1))
    o_ref[...] = jnp.maximum(hc / jnp.sqrt(1e-5 + s), 0.0)


_fin_call = pl.pallas_call(
    _fin_body,
    out_shape=jax.ShapeDtypeStruct((N, D), jnp.float32),
)


def kernel(x, edge_index, Wl0, Wr0, att0, b0, Wl1, Wr1, att1, b1):
    src = edge_index[0].reshape(NW, NCHUNKS, CHUNK)
    dst = edge_index[1].reshape(NW, NCHUNKS, CHUNK)
    h = x
    for Wl, Wr, att, b in ((Wl0, Wr0, att0, b0), (Wl1, Wr1, att1, b1)):
        xl, xr = _mm_call(h, Wl, Wr)
        out_part, den_part = _make_edge_call()(
            xl, xr, src, dst, att.reshape(D))
        h = _fin_call(out_part, den_part, b)
    return h


# speedup vs baseline: 1.0046x; 1.0046x over previous
import functools

import jax
import jax.numpy as jnp
from jax import lax
from jax.experimental import pallas as pl
from jax.experimental.pallas import tpu as pltpu
from jax.experimental.pallas import tpu_sc as plsc

N = 10000
E = 320000
D = 128
NEG_SLOPE = 0.2

NC = 2
NS = 16
NW = NC * NS
EPW = E // NW
CHUNK = 40
NCHUNKS = EPW // CHUNK
ROWS_PER_TILE = 624
TAIL_ROW0 = NS * ROWS_PER_TILE
TAIL_ROWS = N - TAIL_ROW0


def _mm_body(h_ref, wl_ref, wr_ref, xl_ref, xr_ref):
    h = h_ref[...]
    xl_ref[...] = jnp.dot(h, wl_ref[...], preferred_element_type=jnp.float32)
    xr_ref[...] = jnp.dot(h, wr_ref[...], preferred_element_type=jnp.float32)


_mm_call = pl.pallas_call(
    _mm_body,
    out_shape=[jax.ShapeDtypeStruct((N, D), jnp.float32)] * 2,
)


def _edge_body(xl_hbm, xr_hbm, src_hbm, dst_hbm, att_hbm,
               out_hbm, den_hbm,
               xlrows, xrrows, srcv, dstv, denv, attv, exb,
               acc_sh, sem1, sem2, sem3, sem4):
    cid = lax.axis_index("c")
    sid = lax.axis_index("s")
    wid = cid * NS + sid

    pltpu.sync_copy(att_hbm, attv)
    att_vs = [attv[pl.ds(16 * k, 16)] for k in range(8)]

    zero16 = jnp.zeros((16,), jnp.float32)
    lane = lax.iota(jnp.int32, 16)
    lane0 = lane == 0

    def zden_body(j, _):
        denv[pl.ds(16 * j, 16)] = zero16
        return 0

    lax.fori_loop(0, N // 16, zden_body, 0)

    def zrow_body(r, _):
        for k in range(8):
            xlrows[0, r, pl.ds(16 * k, 16)] = zero16
        return 0

    lax.fori_loop(0, CHUNK, zrow_body, 0)
    r0 = sid * ROWS_PER_TILE

    def zcopy_body(j, _):
        pltpu.sync_copy(xlrows.at[0],
                        acc_sh.at[pl.ds(r0 + j * CHUNK, CHUNK)])
        return 0

    lax.fori_loop(0, 15, zcopy_body, 0)
    pltpu.sync_copy(xlrows.at[0, pl.ds(0, 24)],
                    acc_sh.at[pl.ds(r0 + 600, 24)])

    @pl.when(sid == NS - 1)
    def _zero_tail():
        pltpu.sync_copy(xlrows.at[0, pl.ds(0, TAIL_ROWS)],
                        acc_sh.at[pl.ds(TAIL_ROW0, TAIL_ROWS)])

    plsc.subcore_barrier()

    pltpu.sync_copy(src_hbm.at[wid, 0], srcv.at[0])
    pltpu.sync_copy(dst_hbm.at[wid, 0], dstv.at[0])
    pltpu.sync_copy(src_hbm.at[wid, 1], srcv.at[1])
    pltpu.sync_copy(dst_hbm.at[wid, 1], dstv.at[1])
    pltpu.async_copy(xl_hbm.at[srcv.at[0]], xlrows.at[0], sem1)
    pltpu.async_copy(xr_hbm.at[dstv.at[0]], xrrows.at[0], sem2)

    def chunk_body(c, _):
        p = lax.rem(c, 2)
        slot = lax.rem(c, 3)
        slot1 = lax.rem(c + 1, 3)
        slot2 = lax.rem(c + 2, 3)

        @pl.when(c + 2 < NCHUNKS)
        def _idx_prefetch():
            pltpu.async_copy(src_hbm.at[wid, c + 2], srcv.at[slot2], sem4)
            pltpu.async_copy(dst_hbm.at[wid, c + 2], dstv.at[slot2], sem4)

        pltpu.make_async_copy(
            xl_hbm.at[srcv.at[slot]], xlrows.at[p], sem1).wait()
        pltpu.make_async_copy(
            xr_hbm.at[dstv.at[slot]], xrrows.at[p], sem2).wait()

        @pl.when(c > 0)
        def _scatter_wait():
            pltpu.make_async_copy(
                xlrows.at[1 - p], acc_sh.at[dstv.at[lax.rem(c + 2, 3)]],
                sem3).wait()

        @pl.when(c + 1 < NCHUNKS)
        def _row_prefetch():
            @pl.when(c > 0)
            def _idx_wait():
                pltpu.make_async_copy(
                    src_hbm.at[wid, c + 1], srcv.at[slot1], sem4).wait()
                pltpu.make_async_copy(
                    dst_hbm.at[wid, c + 1], dstv.at[slot1], sem4).wait()

            pltpu.async_copy(xl_hbm.at[srcv.at[slot1]], xlrows.at[1 - p], sem1)
            pltpu.async_copy(xr_hbm.at[dstv.at[slot1]], xrrows.at[1 - p], sem2)

        @plsc.parallel_loop(0, CHUNK, unroll=2)
        def edge_body(e):
            acc = jnp.zeros((16,), jnp.float32)
            xls = []
            for k in range(8):
                xlk = xlrows[p, e, pl.ds(16 * k, 16)]
                xls.append(xlk)
                m = xlk + xrrows[p, e, pl.ds(16 * k, 16)]
                m = jnp.maximum(m, NEG_SLOPE * m)
                acc = acc + m * att_vs[k]
            for s in (1, 2, 4, 8):
                acc = acc + lax.gather(
                    acc, (lane ^ s)[:, None],
                    lax.GatherDimensionNumbers(
                        offset_dims=(), collapsed_slice_dims=(0,),
                        start_index_map=(0,)),
                    slice_sizes=(1,),
                    mode=lax.GatherScatterMode.PROMISE_IN_BOUNDS)
            ex16 = jnp.exp(acc)
            plsc.store_scatter(
                exb, [jnp.full((16,), e, jnp.int32)], ex16, mask=lane0)
            for k in range(8):
                xlrows[p, e, pl.ds(16 * k, 16)] = xls[k] * ex16

        for off, msk in ((0, None), (16, None), (24, lane >= 8)):
            ex16 = exb[pl.ds(off, 16)]
            dst16 = dstv[slot, pl.ds(off, 16)]
            plsc.addupdate_scatter(denv, [dst16], ex16, mask=msk)

        pltpu.async_copy(xlrows.at[p], acc_sh.at[dstv.at[slot]], sem3,
                         add=True)
        return 0

    lax.fori_loop(0, NCHUNKS, chunk_body, 0)

    pltpu.make_async_copy(
        xlrows.at[(NCHUNKS - 1) % 2],
        acc_sh.at[dstv.at[(NCHUNKS - 1) % 3]], sem3).wait()

    pltpu.sync_copy(denv, den_hbm.at[wid])

    plsc.subcore_barrier()

    def fcopy_body(j, _):
        rr = r0 + j * CHUNK
        pltpu.sync_copy(acc_sh.at[pl.ds(rr, CHUNK)], xlrows.at[0])
        pltpu.sync_copy(xlrows.at[0], out_hbm.at[cid, pl.ds(rr, CHUNK)])
        return 0

    lax.fori_loop(0, 15, fcopy_body, 0)
    pltpu.sync_copy(acc_sh.at[pl.ds(r0 + 600, 24)],
                    xlrows.at[0, pl.ds(0, 24)])
    pltpu.sync_copy(xlrows.at[0, pl.ds(0, 24)],
                    out_hbm.at[cid, pl.ds(r0 + 600, 24)])

    @pl.when(sid == NS - 1)
    def _flush_tail():
        pltpu.sync_copy(acc_sh.at[pl.ds(TAIL_ROW0, TAIL_ROWS)],
                        xlrows.at[0, pl.ds(0, TAIL_ROWS)])
        pltpu.sync_copy(xlrows.at[0, pl.ds(0, TAIL_ROWS)],
                        out_hbm.at[cid, pl.ds(TAIL_ROW0, TAIL_ROWS)])


@functools.cache
def _make_edge_call():
  return pl.kernel(
    _edge_body,
    out_type=[
        jax.ShapeDtypeStruct((NC, N, D), jnp.float32),
        jax.ShapeDtypeStruct((NW, N), jnp.float32),
    ],
    mesh=plsc.VectorSubcoreMesh(
        core_axis_name="c", subcore_axis_name="s", num_cores=NC,
        num_subcores=NS),
    compiler_params=pltpu.CompilerParams(needs_layout_passes=False),
    scratch_types=[
        pltpu.VMEM((2, CHUNK, D), jnp.float32),
        pltpu.VMEM((2, CHUNK, D), jnp.float32),
        pltpu.VMEM((3, CHUNK), jnp.int32),
        pltpu.VMEM((3, CHUNK), jnp.int32),
        pltpu.VMEM((N,), jnp.float32),
        pltpu.VMEM((D,), jnp.float32),
        pltpu.VMEM((CHUNK,), jnp.float32),
        pltpu.VMEM_SHARED((N, D), jnp.float32),
        pltpu.SemaphoreType.DMA,
        pltpu.SemaphoreType.DMA,
        pltpu.SemaphoreType.DMA,
        pltpu.SemaphoreType.DMA,
    ],
  )


def _fin_body(op_ref, dp_ref, b_ref, o_ref):
    num = op_ref[0] + op_ref[1]
    den = jnp.sum(dp_ref[...], axis=0)[:, None]
    h0 = num / (den + 1e-16) + b_ref[...][None, :]
    mu = jnp.mean(h0, axis=0, keepdims=True)
    hc = h0 - mu
    s = jnp.mean(jnp.sum(hc * hc, axis=-1))
    o_ref[...] = jnp.maximum(hc / jnp.sqrt(1e-5 + s), 0.0)


_fin_call = pl.pallas_call(
    _fin_body,
    out_shape=jax.ShapeDtypeStruct((N, D), jnp.float32),
)


def kernel(x, edge_index, Wl0, Wr0, att0, b0, Wl1, Wr1, att1, b1):
    src = edge_index[0].reshape(NW, NCHUNKS, CHUNK)
    dst = edge_index[1].reshape(NW, NCHUNKS, CHUNK)
    h = x
    for Wl, Wr, att, b in ((Wl0, Wr0, att0, b0), (Wl1, Wr1, att1, b1)):
        xl, xr = _mm_call(h, Wl, Wr)
        out_part, den_part = _make_edge_call()(
            xl, xr, src, dst, att.reshape(D))
        h = _fin_call(out_part, den_part, b)
    return h
